# baseline (speedup 1.0000x reference)
import jax
import jax.numpy as jnp
from jax import lax
from jax.experimental import pallas as pl
from jax.experimental.pallas import tpu as pltpu


def _fused_kernel(prem_ref, hyp_ref,
                  emb_ref,
                  w_ih_ref, w_hh_ref, b_ref,
                  w1_ref, b1_ref,
                  w2_ref, b2_ref,
                  w3_ref, b3_ref,
                  out_ref,
                  x_buf,
                  w_ih_v, w_hh_v, b_v, w1_v, b1_v, w2_v, b2_v, w3_v, b3_v,
                  dma_sem, w_sem):
    B, S = prem_ref.shape
    E = w_ih_ref.shape[0]
    H = w_hh_ref.shape[0]
    B2 = 2 * B
    M = S * B2
    H4 = 4 * H

    w_pairs = ((w_ih_ref, w_ih_v), (w_hh_ref, w_hh_v), (b_ref, b_v),
               (w1_ref, w1_v), (b1_ref, b1_v), (w2_ref, w2_v),
               (b2_ref, b2_v), (w3_ref, w3_v), (b3_ref, b3_v))
    for src, dst in w_pairs:
        pltpu.make_async_copy(src, dst, w_sem).start()

    for t in range(S - 1, -1, -1):
        for r in range(B2):
            tok = prem_ref[r, t] if r < B else hyp_ref[r - B, t]
            pltpu.make_async_copy(emb_ref.at[pl.ds(tok, 1), :],
                                  x_buf.at[t * B2 + r], dma_sem).start()

    for src, dst in w_pairs:
        pltpu.make_async_copy(src, dst, w_sem).wait()

    gate_q = lax.broadcasted_iota(jnp.int32, (1, H4), 1) // H
    gscale = jnp.where(gate_q == 2, 1.0, 0.5).astype(jnp.float32)
    w_hh_s = w_hh_v[...] * gscale
    b_s = b_v[...] * gscale
    w_ih_s = w_ih_v[...] * gscale

    pltpu.make_async_copy(emb_ref.at[pl.ds(0, M), :],
                          x_buf.at[pl.ds(0, M), 0], dma_sem).wait()

    x = x_buf[:, 0, :]
    gx = (jnp.dot(x, w_ih_s, preferred_element_type=jnp.float32)
          + b_s)

    def gates(z):
        th = jnp.tanh(z)
        return (th[:, :H], th[:, H:2 * H], th[:, 2 * H:3 * H], th[:, 3 * H:])

    i_g, _, g_g, o_g = gates(gx[(S - 1) * B2:S * B2, :])
    c = (0.5 * i_g + 0.5) * g_g
    h = (0.5 * o_g + 0.5) * jnp.tanh(c)
    for t in range(S - 2, -1, -1):
        z = gx[t * B2:(t + 1) * B2, :] + jnp.dot(
            h, w_hh_s, preferred_element_type=jnp.float32)
        i_g, f_g, g_g, o_g = gates(z)
        c = (0.5 * f_g + 0.5) * c + (0.5 * i_g + 0.5) * g_g
        h = (0.5 * o_g + 0.5) * jnp.tanh(c)

    y = jnp.maximum(
        jnp.dot(h[:B, :], w1_v[:H, :], preferred_element_type=jnp.float32)
        + jnp.dot(h[B:, :], w1_v[H:, :], preferred_element_type=jnp.float32)
        + b1_v[...], 0.0)
    y = jnp.maximum(
        jnp.dot(y, w2_v[...], preferred_element_type=jnp.float32)
        + b2_v[...], 0.0)
    y = jnp.maximum(
        jnp.dot(y, w3_v[...], preferred_element_type=jnp.float32)
        + b3_v[...], 0.0)
    out_ref[...] = y.astype(out_ref.dtype)


@jax.jit
def _forward(embedding, w_ih_rev, w_hh_rev, b_lstm_rev,
             w1, b1, w2, b2, w3, b3, premise, hypothesis):
    B, S = premise.shape
    V, E = embedding.shape
    H = w_hh_rev.shape[0]
    C = w3.shape[1]
    M = S * 2 * B

    dense = (w_ih_rev, w_hh_rev, b_lstm_rev, w1, b1, w2, b2, w3, b3)

    smem = pl.BlockSpec(memory_space=pltpu.MemorySpace.SMEM)
    hbm = pl.BlockSpec(memory_space=pltpu.MemorySpace.HBM)
    return pl.pallas_call(
        _fused_kernel,
        out_shape=jax.ShapeDtypeStruct((B, C), jnp.float32),
        grid=(1,),
        in_specs=[smem, smem] + [hbm] * 10,
        out_specs=pl.BlockSpec((B, C), lambda i: (0, 0)),
        scratch_shapes=[pltpu.VMEM((M, 1, E), jnp.float32)]
                       + [pltpu.VMEM(a.shape, jnp.float32) for a in dense]
                       + [pltpu.SemaphoreType.DMA, pltpu.SemaphoreType.DMA],
        compiler_params=pltpu.CompilerParams(
            dimension_semantics=("arbitrary",)),
    )(premise, hypothesis, embedding, *dense)


def kernel(embedding, w_ih_rev, w_hh_rev, b_lstm_rev,
           w1, b1, w2, b2, w3, b3, premise, hypothesis):
    return _forward(embedding, w_ih_rev, w_hh_rev, b_lstm_rev,
                    w1, b1, w2, b2, w3, b3, premise, hypothesis)
